# baseline (speedup 1.0000x reference)
import functools

import jax
import jax.numpy as jnp
from jax import lax
from jax.experimental import pallas as pl
from jax.experimental.pallas import tpu as pltpu
from jax.experimental.pallas import tpu_sc as plsc

R, C = 100, 16384
NC, NS = 2, 16
NW = NC * NS
COLS_W = C // NW
NBUF = 2
COLS_C = 256
NCHUNK = COLS_W // COLS_C

_mesh = plsc.VectorSubcoreMesh(
    core_axis_name="c", subcore_axis_name="s", num_cores=NC, num_subcores=NS
)


@functools.partial(
    pl.kernel,
    out_type=jax.ShapeDtypeStruct((R, C), jnp.int32),
    mesh=_mesh,
    compiler_params=pltpu.CompilerParams(use_tc_tiling_on_sc=True),
    scratch_types=[
        pltpu.VMEM((16,), jnp.int32),
        pltpu.VMEM((R, COLS_C), jnp.int32), pltpu.VMEM((R, COLS_C), jnp.int32),
        pltpu.VMEM((R, COLS_C), jnp.int32), pltpu.VMEM((R, COLS_C), jnp.int32),
        pltpu.SemaphoreType.DMA, pltpu.SemaphoreType.DMA,
        pltpu.SemaphoreType.DMA, pltpu.SemaphoreType.DMA,
    ],
)
def _sc_lookup(in_hbm, tg_hbm, tab_hbm, out_hbm,
               tab_v, in0, in1, tg0, tg1, si0, si1, so0, so1):
    wid = lax.axis_index("s") * NC + lax.axis_index("c")
    base = wid * COLS_W
    pltpu.sync_copy(tab_hbm, tab_v)
    tab = tab_v[...]

    bufs = ((in0, tg0, si0, so0), (in1, tg1, si1, so1))

    for b in range(NBUF):
        off = base + b * COLS_C
        in_v, tg_v, sem_i, _ = bufs[b]
        pltpu.async_copy(in_hbm.at[:, pl.ds(off, COLS_C)], in_v, sem_i)
        pltpu.async_copy(tg_hbm.at[:, pl.ds(off, COLS_C)], tg_v, sem_i)

    for ci in range(NCHUNK):
        in_v, tg_v, sem_i, sem_o = bufs[ci % NBUF]
        off = base + ci * COLS_C
        pltpu.make_async_copy(in_hbm.at[:, pl.ds(off, COLS_C)], in_v, sem_i).wait()
        pltpu.make_async_copy(tg_hbm.at[:, pl.ds(off, COLS_C)], tg_v, sem_i).wait()
        if ci >= NBUF:
            prev = base + (ci - NBUF) * COLS_C
            pltpu.make_async_copy(
                in_v, out_hbm.at[:, pl.ds(prev, COLS_C)], sem_o).wait()

        @plsc.parallel_loop(0, R, 1, unroll=4)
        def _row(r):
            for c in range(0, COLS_C, 16):
                s = (r, pl.ds(c, 16))
                idx = in_v[s] * 4 + tg_v[s]
                in_v[s] = tab.at[idx].get(mode="promise_in_bounds")

        pltpu.async_copy(in_v, out_hbm.at[:, pl.ds(off, COLS_C)], sem_o)
        if ci + NBUF < NCHUNK:
            pltpu.make_async_copy(
                in_v, out_hbm.at[:, pl.ds(off, COLS_C)], sem_o).wait()
            noff = base + (ci + NBUF) * COLS_C
            pltpu.async_copy(in_hbm.at[:, pl.ds(noff, COLS_C)], in_v, sem_i)
            pltpu.async_copy(tg_hbm.at[:, pl.ds(noff, COLS_C)], tg_v, sem_i)

    for ci in range(max(NCHUNK - NBUF, 0), NCHUNK):
        in_v, _, _, sem_o = bufs[ci % NBUF]
        off = base + ci * COLS_C
        pltpu.make_async_copy(in_v, out_hbm.at[:, pl.ds(off, COLS_C)], sem_o).wait()


def kernel(input_, target, sim_lookup):
    out_bits = _sc_lookup(
        input_.T.astype(jnp.int32),
        target.T.astype(jnp.int32),
        lax.bitcast_convert_type(sim_lookup.astype(jnp.float32), jnp.int32),
    )
    return lax.bitcast_convert_type(out_bits.T, jnp.float32)


# speedup vs baseline: 1.0214x; 1.0214x over previous
import functools

import jax
import jax.numpy as jnp
from jax import lax
from jax.experimental import pallas as pl
from jax.experimental.pallas import tpu as pltpu
from jax.experimental.pallas import tpu_sc as plsc

R, C = 100, 16384
NC, NS = 2, 16
NW = NC * NS
COLS_W = C // NW
NBUF = 2
COLS_C = 128
NCHUNK = COLS_W // COLS_C

_mesh = plsc.VectorSubcoreMesh(
    core_axis_name="c", subcore_axis_name="s", num_cores=NC, num_subcores=NS
)


@functools.partial(
    pl.kernel,
    out_type=jax.ShapeDtypeStruct((R, C), jnp.int32),
    mesh=_mesh,
    compiler_params=pltpu.CompilerParams(use_tc_tiling_on_sc=True),
    scratch_types=[
        pltpu.VMEM((16,), jnp.int32),
        pltpu.VMEM((R, COLS_C), jnp.int32), pltpu.VMEM((R, COLS_C), jnp.int32),
        pltpu.VMEM((R, COLS_C), jnp.int32), pltpu.VMEM((R, COLS_C), jnp.int32),
        pltpu.VMEM((R, COLS_C), jnp.int32), pltpu.VMEM((R, COLS_C), jnp.int32),
        pltpu.SemaphoreType.DMA, pltpu.SemaphoreType.DMA,
        pltpu.SemaphoreType.DMA, pltpu.SemaphoreType.DMA,
    ],
)
def _sc_lookup(in_hbm, tg_hbm, tab_hbm, out_hbm,
               tab_v, in0, in1, tg0, tg1, out0, out1, si0, si1, so0, so1):
    wid = lax.axis_index("s") * NC + lax.axis_index("c")
    base = wid * COLS_W
    pltpu.sync_copy(tab_hbm, tab_v)
    tab = tab_v[...]

    bufs = ((in0, tg0, out0, si0, so0), (in1, tg1, out1, si1, so1))

    for b in range(NBUF):
        off = base + b * COLS_C
        in_v, tg_v, _, sem_i, _ = bufs[b]
        pltpu.async_copy(in_hbm.at[:, pl.ds(off, COLS_C)], in_v, sem_i)
        pltpu.async_copy(tg_hbm.at[:, pl.ds(off, COLS_C)], tg_v, sem_i)

    for ci in range(NCHUNK):
        in_v, tg_v, out_v, sem_i, sem_o = bufs[ci % NBUF]
        off = base + ci * COLS_C
        pltpu.make_async_copy(in_hbm.at[:, pl.ds(off, COLS_C)], in_v, sem_i).wait()
        pltpu.make_async_copy(tg_hbm.at[:, pl.ds(off, COLS_C)], tg_v, sem_i).wait()
        if ci >= NBUF:
            prev = base + (ci - NBUF) * COLS_C
            pltpu.make_async_copy(
                out_v, out_hbm.at[:, pl.ds(prev, COLS_C)], sem_o).wait()

        @plsc.parallel_loop(0, R, 1, unroll=4)
        def _row(r):
            for c in range(0, COLS_C, 16):
                s = (r, pl.ds(c, 16))
                idx = in_v[s] * 4 + tg_v[s]
                out_v[s] = tab.at[idx].get(mode="promise_in_bounds")

        pltpu.async_copy(out_v, out_hbm.at[:, pl.ds(off, COLS_C)], sem_o)
        if ci + NBUF < NCHUNK:
            noff = base + (ci + NBUF) * COLS_C
            pltpu.async_copy(in_hbm.at[:, pl.ds(noff, COLS_C)], in_v, sem_i)
            pltpu.async_copy(tg_hbm.at[:, pl.ds(noff, COLS_C)], tg_v, sem_i)

    for ci in range(max(NCHUNK - NBUF, 0), NCHUNK):
        _, _, out_v, _, sem_o = bufs[ci % NBUF]
        off = base + ci * COLS_C
        pltpu.make_async_copy(out_v, out_hbm.at[:, pl.ds(off, COLS_C)], sem_o).wait()


def kernel(input_, target, sim_lookup):
    out_bits = _sc_lookup(
        input_.T.astype(jnp.int32),
        target.T.astype(jnp.int32),
        lax.bitcast_convert_type(sim_lookup.astype(jnp.float32), jnp.int32),
    )
    return lax.bitcast_convert_type(out_bits.T, jnp.float32)


# speedup vs baseline: 1.0324x; 1.0107x over previous
import functools

import jax
import jax.numpy as jnp
from jax import lax
from jax.experimental import pallas as pl
from jax.experimental.pallas import tpu as pltpu
from jax.experimental.pallas import tpu_sc as plsc

R, C = 100, 16384
NC, NS = 2, 16
NW = NC * NS
COLS_W = C // NW
NBUF = 2
COLS_C = 128
NCHUNK = COLS_W // COLS_C

_mesh = plsc.VectorSubcoreMesh(
    core_axis_name="c", subcore_axis_name="s", num_cores=NC, num_subcores=NS
)


@functools.partial(
    pl.kernel,
    out_type=jax.ShapeDtypeStruct((R, C), jnp.int32),
    mesh=_mesh,
    compiler_params=pltpu.CompilerParams(use_tc_tiling_on_sc=True),
    scratch_types=[
        pltpu.VMEM((16,), jnp.int32),
        pltpu.VMEM((R, COLS_C), jnp.int32), pltpu.VMEM((R, COLS_C), jnp.int32),
        pltpu.VMEM((R, COLS_C), jnp.int32), pltpu.VMEM((R, COLS_C), jnp.int32),
        pltpu.VMEM((R, COLS_C), jnp.int32), pltpu.VMEM((R, COLS_C), jnp.int32),
        pltpu.SemaphoreType.DMA, pltpu.SemaphoreType.DMA,
        pltpu.SemaphoreType.DMA, pltpu.SemaphoreType.DMA,
    ],
)
def _sc_lookup(in_hbm, tg_hbm, tab_hbm, out_hbm,
               tab_v, in0, in1, tg0, tg1, out0, out1, si0, si1, so0, so1):
    wid = lax.axis_index("s") * NC + lax.axis_index("c")
    base = wid * COLS_W
    pltpu.sync_copy(tab_hbm, tab_v)
    tab = tab_v[...]

    bufs = ((in0, tg0, out0, si0, so0), (in1, tg1, out1, si1, so1))

    for b in range(NBUF):
        off = base + b * COLS_C
        in_v, tg_v, _, sem_i, _ = bufs[b]
        pltpu.async_copy(in_hbm.at[:, pl.ds(off, COLS_C)], in_v, sem_i)
        pltpu.async_copy(tg_hbm.at[:, pl.ds(off, COLS_C)], tg_v, sem_i)

    for ci in range(NCHUNK):
        in_v, tg_v, out_v, sem_i, sem_o = bufs[ci % NBUF]
        off = base + ci * COLS_C
        pltpu.make_async_copy(in_hbm.at[:, pl.ds(off, COLS_C)], in_v, sem_i).wait()
        pltpu.make_async_copy(tg_hbm.at[:, pl.ds(off, COLS_C)], tg_v, sem_i).wait()
        if ci >= NBUF:
            prev = base + (ci - NBUF) * COLS_C
            pltpu.make_async_copy(
                out_v, out_hbm.at[:, pl.ds(prev, COLS_C)], sem_o).wait()

        @plsc.parallel_loop(0, R, 1, unroll=2)
        def _row(r):
            for c in range(0, COLS_C, 16):
                s = (r, pl.ds(c, 16))
                idx = in_v[s] * 4 + tg_v[s]
                out_v[s] = tab.at[idx].get(mode="promise_in_bounds")

        pltpu.async_copy(out_v, out_hbm.at[:, pl.ds(off, COLS_C)], sem_o)
        if ci + NBUF < NCHUNK:
            noff = base + (ci + NBUF) * COLS_C
            pltpu.async_copy(in_hbm.at[:, pl.ds(noff, COLS_C)], in_v, sem_i)
            pltpu.async_copy(tg_hbm.at[:, pl.ds(noff, COLS_C)], tg_v, sem_i)

    for ci in range(max(NCHUNK - NBUF, 0), NCHUNK):
        _, _, out_v, _, sem_o = bufs[ci % NBUF]
        off = base + ci * COLS_C
        pltpu.make_async_copy(out_v, out_hbm.at[:, pl.ds(off, COLS_C)], sem_o).wait()


def kernel(input_, target, sim_lookup):
    out_bits = _sc_lookup(
        input_.T.astype(jnp.int32),
        target.T.astype(jnp.int32),
        lax.bitcast_convert_type(sim_lookup.astype(jnp.float32), jnp.int32),
    )
    return lax.bitcast_convert_type(out_bits.T, jnp.float32)


# speedup vs baseline: 1.2235x; 1.1852x over previous
import functools

import jax
import jax.numpy as jnp
from jax import lax
from jax.experimental import pallas as pl
from jax.experimental.pallas import tpu as pltpu
from jax.experimental.pallas import tpu_sc as plsc

R, C = 100, 16384
NC, NS = 2, 16
NW = NC * NS
COLS_W = C // NW
NBUF = 2
COLS_C = 128
NCHUNK = COLS_W // COLS_C

_mesh = plsc.VectorSubcoreMesh(
    core_axis_name="c", subcore_axis_name="s", num_cores=NC, num_subcores=NS
)


@functools.partial(
    pl.kernel,
    out_type=jax.ShapeDtypeStruct((R, C), jnp.float32),
    mesh=_mesh,
    compiler_params=pltpu.CompilerParams(use_tc_tiling_on_sc=True),
    scratch_types=[
        pltpu.VMEM((16,), jnp.float32),
        pltpu.VMEM((R, COLS_C), jnp.int32), pltpu.VMEM((R, COLS_C), jnp.int32),
        pltpu.VMEM((R, COLS_C), jnp.int32), pltpu.VMEM((R, COLS_C), jnp.int32),
        pltpu.VMEM((R, COLS_C), jnp.float32), pltpu.VMEM((R, COLS_C), jnp.float32),
        pltpu.SemaphoreType.DMA, pltpu.SemaphoreType.DMA,
        pltpu.SemaphoreType.DMA, pltpu.SemaphoreType.DMA,
    ],
)
def _sc_lookup(in_hbm, tg_hbm, tab_hbm, out_hbm,
               tab_v, in0, in1, tg0, tg1, out0, out1, si0, si1, so0, so1):
    wid = lax.axis_index("s") * NC + lax.axis_index("c")
    base = wid * COLS_W
    pltpu.sync_copy(tab_hbm, tab_v)
    tab = tab_v[...]

    bufs = ((in0, tg0, out0, si0, so0), (in1, tg1, out1, si1, so1))

    for b in range(NBUF):
        off = base + b * COLS_C
        in_v, tg_v, _, sem_i, _ = bufs[b]
        pltpu.async_copy(in_hbm.at[:, pl.ds(off, COLS_C)], in_v, sem_i)
        pltpu.async_copy(tg_hbm.at[:, pl.ds(off, COLS_C)], tg_v, sem_i)

    for ci in range(NCHUNK):
        in_v, tg_v, out_v, sem_i, sem_o = bufs[ci % NBUF]
        off = base + ci * COLS_C
        pltpu.make_async_copy(in_hbm.at[:, pl.ds(off, COLS_C)], in_v, sem_i).wait()
        pltpu.make_async_copy(tg_hbm.at[:, pl.ds(off, COLS_C)], tg_v, sem_i).wait()
        if ci >= NBUF:
            prev = base + (ci - NBUF) * COLS_C
            pltpu.make_async_copy(
                out_v, out_hbm.at[:, pl.ds(prev, COLS_C)], sem_o).wait()

        @plsc.parallel_loop(0, R, 1, unroll=2)
        def _row(r):
            for c in range(0, COLS_C, 16):
                s = (r, pl.ds(c, 16))
                idx = in_v[s] * 4 + tg_v[s]
                out_v[s] = tab.at[idx].get(mode="promise_in_bounds")

        pltpu.async_copy(out_v, out_hbm.at[:, pl.ds(off, COLS_C)], sem_o)
        if ci + NBUF < NCHUNK:
            noff = base + (ci + NBUF) * COLS_C
            pltpu.async_copy(in_hbm.at[:, pl.ds(noff, COLS_C)], in_v, sem_i)
            pltpu.async_copy(tg_hbm.at[:, pl.ds(noff, COLS_C)], tg_v, sem_i)

    for ci in range(max(NCHUNK - NBUF, 0), NCHUNK):
        _, _, out_v, _, sem_o = bufs[ci % NBUF]
        off = base + ci * COLS_C
        pltpu.make_async_copy(out_v, out_hbm.at[:, pl.ds(off, COLS_C)], sem_o).wait()


def kernel(input_, target, sim_lookup):
    out_t = _sc_lookup(
        input_.T.astype(jnp.int32),
        target.T.astype(jnp.int32),
        sim_lookup.astype(jnp.float32),
    )
    return out_t.T


# speedup vs baseline: 1.2792x; 1.0455x over previous
import functools

import jax
import jax.numpy as jnp
from jax import lax
from jax.experimental import pallas as pl
from jax.experimental.pallas import tpu as pltpu
from jax.experimental.pallas import tpu_sc as plsc

R, C = 100, 16384
NC, NS = 2, 16
NW = NC * NS
COLS_W = C // NW
NBUF = 2
COLS_C = 128
NCHUNK = COLS_W // COLS_C

_mesh = plsc.VectorSubcoreMesh(
    core_axis_name="c", subcore_axis_name="s", num_cores=NC, num_subcores=NS
)


@functools.partial(
    pl.kernel,
    out_type=jax.ShapeDtypeStruct((R, C), jnp.float32),
    mesh=_mesh,
    compiler_params=pltpu.CompilerParams(
        use_tc_tiling_on_sc=True,
        disable_bounds_checks=True,
        disable_semaphore_checks=True,
    ),
    scratch_types=[
        pltpu.VMEM((16,), jnp.float32),
        pltpu.VMEM((R, COLS_C), jnp.int32), pltpu.VMEM((R, COLS_C), jnp.int32),
        pltpu.VMEM((R, COLS_C), jnp.int32), pltpu.VMEM((R, COLS_C), jnp.int32),
        pltpu.VMEM((R, COLS_C), jnp.float32), pltpu.VMEM((R, COLS_C), jnp.float32),
        pltpu.SemaphoreType.DMA, pltpu.SemaphoreType.DMA,
        pltpu.SemaphoreType.DMA, pltpu.SemaphoreType.DMA,
    ],
)
def _sc_lookup(in_hbm, tg_hbm, tab_hbm, out_hbm,
               tab_v, in0, in1, tg0, tg1, out0, out1, si0, si1, so0, so1):
    wid = lax.axis_index("s") * NC + lax.axis_index("c")
    base = wid * COLS_W

    bufs = ((in0, tg0, out0, si0, so0), (in1, tg1, out1, si1, so1))

    for b in range(NBUF):
        off = base + b * COLS_C
        in_v, tg_v, _, sem_i, _ = bufs[b]
        pltpu.async_copy(in_hbm.at[:, pl.ds(off, COLS_C)], in_v, sem_i)
        pltpu.async_copy(tg_hbm.at[:, pl.ds(off, COLS_C)], tg_v, sem_i)

    pltpu.sync_copy(tab_hbm, tab_v)
    tab = tab_v[...]

    for ci in range(NCHUNK):
        in_v, tg_v, out_v, sem_i, sem_o = bufs[ci % NBUF]
        off = base + ci * COLS_C
        pltpu.make_async_copy(in_hbm.at[:, pl.ds(off, COLS_C)], in_v, sem_i).wait()
        pltpu.make_async_copy(tg_hbm.at[:, pl.ds(off, COLS_C)], tg_v, sem_i).wait()
        if ci >= NBUF:
            prev = base + (ci - NBUF) * COLS_C
            pltpu.make_async_copy(
                out_v, out_hbm.at[:, pl.ds(prev, COLS_C)], sem_o).wait()

        @plsc.parallel_loop(0, R, 1, unroll=2)
        def _row(r):
            for c in range(0, COLS_C, 16):
                s = (r, pl.ds(c, 16))
                idx = in_v[s] * 4 + tg_v[s]
                out_v[s] = tab.at[idx].get(mode="promise_in_bounds")

        pltpu.async_copy(out_v, out_hbm.at[:, pl.ds(off, COLS_C)], sem_o)
        if ci + NBUF < NCHUNK:
            noff = base + (ci + NBUF) * COLS_C
            pltpu.async_copy(in_hbm.at[:, pl.ds(noff, COLS_C)], in_v, sem_i)
            pltpu.async_copy(tg_hbm.at[:, pl.ds(noff, COLS_C)], tg_v, sem_i)

    for ci in range(max(NCHUNK - NBUF, 0), NCHUNK):
        _, _, out_v, _, sem_o = bufs[ci % NBUF]
        off = base + ci * COLS_C
        pltpu.make_async_copy(out_v, out_hbm.at[:, pl.ds(off, COLS_C)], sem_o).wait()


def kernel(input_, target, sim_lookup):
    out_t = _sc_lookup(
        input_.T.astype(jnp.int32),
        target.T.astype(jnp.int32),
        sim_lookup.astype(jnp.float32),
    )
    return out_t.T


# speedup vs baseline: 1.3176x; 1.0300x over previous
import functools

import jax
import jax.numpy as jnp
from jax import lax
from jax.experimental import pallas as pl
from jax.experimental.pallas import tpu as pltpu
from jax.experimental.pallas import tpu_sc as plsc

R, C = 100, 16384
NC, NS = 2, 16
NW = NC * NS
COLS_W = C // NW
NBUF = 2
COLS_C = 128
NCHUNK = COLS_W // COLS_C

_mesh = plsc.VectorSubcoreMesh(
    core_axis_name="c", subcore_axis_name="s", num_cores=NC, num_subcores=NS
)


@functools.partial(
    pl.kernel,
    out_type=jax.ShapeDtypeStruct((R, C), jnp.float32),
    mesh=_mesh,
    compiler_params=pltpu.CompilerParams(
        use_tc_tiling_on_sc=True,
        disable_bounds_checks=True,
        disable_semaphore_checks=True,
    ),
    scratch_types=[
        pltpu.VMEM((16,), jnp.float32),
        pltpu.VMEM((R, COLS_C), jnp.int32), pltpu.VMEM((R, COLS_C), jnp.int32),
        pltpu.VMEM((R, COLS_C), jnp.int32), pltpu.VMEM((R, COLS_C), jnp.int32),
        pltpu.VMEM((R, COLS_C), jnp.float32), pltpu.VMEM((R, COLS_C), jnp.float32),
        pltpu.SemaphoreType.DMA, pltpu.SemaphoreType.DMA,
        pltpu.SemaphoreType.DMA, pltpu.SemaphoreType.DMA,
    ],
)
def _sc_lookup(in_hbm, tg_hbm, tab_hbm, out_hbm,
               tab_v, in0, in1, tg0, tg1, out0, out1, si0, si1, so0, so1):
    wid = lax.axis_index("s") * NC + lax.axis_index("c")
    base = wid * COLS_W

    bufs = ((in0, tg0, out0, si0, so0), (in1, tg1, out1, si1, so1))

    for b in range(NBUF):
        off = base + b * COLS_C
        in_v, tg_v, _, sem_i, _ = bufs[b]
        pltpu.async_copy(in_hbm.at[:, pl.ds(off, COLS_C)], in_v, sem_i)
        pltpu.async_copy(tg_hbm.at[:, pl.ds(off, COLS_C)], tg_v, sem_i)

    pltpu.sync_copy(tab_hbm, tab_v)
    tab = tab_v[...]

    def round_body(g, _):
        for b in range(NBUF):
            in_v, tg_v, out_v, sem_i, sem_o = bufs[b]
            ci = g * NBUF + b
            off = base + ci * COLS_C
            pltpu.make_async_copy(
                in_hbm.at[:, pl.ds(off, COLS_C)], in_v, sem_i).wait()
            pltpu.make_async_copy(
                tg_hbm.at[:, pl.ds(off, COLS_C)], tg_v, sem_i).wait()

            @pl.when(ci >= NBUF)
            def _():
                prev = off - NBUF * COLS_C
                pltpu.make_async_copy(
                    out_v, out_hbm.at[:, pl.ds(prev, COLS_C)], sem_o).wait()

            @plsc.parallel_loop(0, R, 1, unroll=2)
            def _row(r):
                for c in range(0, COLS_C, 16):
                    s = (r, pl.ds(c, 16))
                    idx = in_v[s] * 4 + tg_v[s]
                    out_v[s] = tab.at[idx].get(mode="promise_in_bounds")

            pltpu.async_copy(out_v, out_hbm.at[:, pl.ds(off, COLS_C)], sem_o)

            @pl.when(ci + NBUF < NCHUNK)
            def _():
                noff = off + NBUF * COLS_C
                pltpu.async_copy(in_hbm.at[:, pl.ds(noff, COLS_C)], in_v, sem_i)
                pltpu.async_copy(tg_hbm.at[:, pl.ds(noff, COLS_C)], tg_v, sem_i)
        return 0

    lax.fori_loop(0, NCHUNK // NBUF, round_body, 0)

    for ci in range(max(NCHUNK - NBUF, 0), NCHUNK):
        _, _, out_v, _, sem_o = bufs[ci % NBUF]
        off = base + ci * COLS_C
        pltpu.make_async_copy(out_v, out_hbm.at[:, pl.ds(off, COLS_C)], sem_o).wait()


def kernel(input_, target, sim_lookup):
    out_t = _sc_lookup(
        input_.T.astype(jnp.int32),
        target.T.astype(jnp.int32),
        sim_lookup.astype(jnp.float32),
    )
    return out_t.T
